# Initial kernel scaffold; baseline (speedup 1.0000x reference)
import functools

import jax
import jax.numpy as jnp
from jax import lax
from jax.experimental import pallas as pl
from jax.experimental.pallas import tpu as pltpu
from jax.experimental.pallas import tpu_sc as plsc

N = 10000
E = 320000
F = 128
TE = 128
HEADS = 4
HD = 32
M = 256
OUT = 128
NITERS = 7
BN_EPS = 1e-5

NC = 2
NS = 16
NW = NC * NS
EPT = E // NW
CH = 400
NCHUNK = EPT // CH
RPT = N // NS
DCH = 125
NDCH = RPT // DCH

BE = 512
NEB = E // BE
BNODE = 400
NNB = N // BNODE

_f32 = jnp.float32


def _mesh():
    return plsc.VectorSubcoreMesh(core_axis_name="c", subcore_axis_name="s")


def _sc_gather(table, ids, width):
    @functools.partial(
        pl.kernel,
        out_type=jax.ShapeDtypeStruct((E, width), _f32),
        mesh=_mesh(),
        scratch_types=[
            pltpu.VMEM((CH,), jnp.int32),
            pltpu.VMEM((CH, width), _f32),
            pltpu.SemaphoreType.DMA,
        ],
    )
    def k(tab_hbm, ids_hbm, out_hbm, idx_v, rows_v, sem):
        wid = lax.axis_index("s") * NC + lax.axis_index("c")
        base = wid * EPT

        @pl.loop(0, NCHUNK)
        def _(i):
            off = base + i * CH
            pltpu.sync_copy(ids_hbm.at[pl.ds(off, CH)], idx_v)
            pltpu.async_copy(tab_hbm.at[idx_v], rows_v, sem).wait()
            pltpu.sync_copy(rows_v, out_hbm.at[pl.ds(off, CH)])

    return k(table, ids)


def _sc_gather_pair(table, ids_a, ids_b, width):
    @functools.partial(
        pl.kernel,
        out_type=(jax.ShapeDtypeStruct((E, width), _f32),
                  jax.ShapeDtypeStruct((E, width), _f32)),
        mesh=_mesh(),
        scratch_types=[
            pltpu.VMEM((CH,), jnp.int32),
            pltpu.VMEM((CH, width), _f32),
            pltpu.VMEM((CH,), jnp.int32),
            pltpu.VMEM((CH, width), _f32),
            pltpu.SemaphoreType.DMA,
            pltpu.SemaphoreType.DMA,
        ],
    )
    def k(tab_hbm, ia_hbm, ib_hbm, oa_hbm, ob_hbm, ia_v, ra_v, ib_v, rb_v,
          sa, sb):
        wid = lax.axis_index("s") * NC + lax.axis_index("c")
        base = wid * EPT

        @pl.loop(0, NCHUNK)
        def _(i):
            off = base + i * CH
            pltpu.sync_copy(ia_hbm.at[pl.ds(off, CH)], ia_v)
            pltpu.sync_copy(ib_hbm.at[pl.ds(off, CH)], ib_v)
            da = pltpu.async_copy(tab_hbm.at[ia_v], ra_v, sa)
            db = pltpu.async_copy(tab_hbm.at[ib_v], rb_v, sb)
            da.wait()
            db.wait()
            pltpu.sync_copy(ra_v, oa_hbm.at[pl.ds(off, CH)])
            pltpu.sync_copy(rb_v, ob_hbm.at[pl.ds(off, CH)])

    return k(table, ids_a, ids_b)


def _sc_scatter_add(vals, ids, zeros, width):
    @functools.partial(
        pl.kernel,
        out_type=jax.ShapeDtypeStruct((NC, N, width), _f32),
        mesh=_mesh(),
        scratch_types=[
            pltpu.VMEM((CH,), jnp.int32),
            pltpu.VMEM((CH, width), _f32),
            pltpu.VMEM((DCH, width), _f32),
            pltpu.VMEM_SHARED((N, width), _f32),
        ],
    )
    def k(vals_hbm, ids_hbm, z_hbm, out_hbm, idx_v, rows_v, dbuf, table):
        cid = lax.axis_index("c")
        sid = lax.axis_index("s")
        wid = sid * NC + cid
        base = wid * EPT
        nbase = sid * RPT

        pltpu.sync_copy(z_hbm.at[pl.ds(0, DCH)], dbuf)

        @pl.loop(0, NDCH)
        def _(j):
            pltpu.sync_copy(dbuf, table.at[pl.ds(nbase + j * DCH, DCH)])

        plsc.subcore_barrier()

        @pl.loop(0, NCHUNK)
        def _(i):
            off = base + i * CH
            pltpu.sync_copy(ids_hbm.at[pl.ds(off, CH)], idx_v)
            pltpu.sync_copy(vals_hbm.at[pl.ds(off, CH)], rows_v)
            pltpu.sync_copy(rows_v, table.at[idx_v], add=True)

        plsc.subcore_barrier()

        @pl.loop(0, NDCH)
        def _(j):
            row = nbase + j * DCH
            pltpu.sync_copy(table.at[pl.ds(row, DCH)], dbuf)
            pltpu.sync_copy(dbuf, out_hbm.at[cid, pl.ds(row, DCH)])

    return k(vals, ids, zeros)


def _sc_degree_counts(ones, src_ids, dst_ids, zeros):
    @functools.partial(
        pl.kernel,
        out_type=jax.ShapeDtypeStruct((2, NC, N, 16), _f32),
        mesh=_mesh(),
        scratch_types=[
            pltpu.VMEM((CH,), jnp.int32),
            pltpu.VMEM((CH, 16), _f32),
            pltpu.VMEM((DCH, 16), _f32),
            pltpu.VMEM_SHARED((N, 16), _f32),
            pltpu.VMEM_SHARED((N, 16), _f32),
        ],
    )
    def k(ones_hbm, src_hbm, dst_hbm, z_hbm, out_hbm, idx_v, rows_v, dbuf,
          tab_s, tab_d):
        cid = lax.axis_index("c")
        sid = lax.axis_index("s")
        wid = sid * NC + cid
        base = wid * EPT
        nbase = sid * RPT

        pltpu.sync_copy(z_hbm.at[pl.ds(0, DCH)], dbuf)

        @pl.loop(0, NDCH)
        def _(j):
            pltpu.sync_copy(dbuf, tab_s.at[pl.ds(nbase + j * DCH, DCH)])
            pltpu.sync_copy(dbuf, tab_d.at[pl.ds(nbase + j * DCH, DCH)])

        plsc.subcore_barrier()
        pltpu.sync_copy(ones_hbm.at[pl.ds(0, CH)], rows_v)

        @pl.loop(0, NCHUNK)
        def _(i):
            off = base + i * CH
            pltpu.sync_copy(src_hbm.at[pl.ds(off, CH)], idx_v)
            pltpu.sync_copy(rows_v, tab_s.at[idx_v], add=True)
            pltpu.sync_copy(dst_hbm.at[pl.ds(off, CH)], idx_v)
            pltpu.sync_copy(rows_v, tab_d.at[idx_v], add=True)

        plsc.subcore_barrier()

        @pl.loop(0, NDCH)
        def _(j):
            row = nbase + j * DCH
            pltpu.sync_copy(tab_s.at[pl.ds(row, DCH)], dbuf)
            pltpu.sync_copy(dbuf, out_hbm.at[0, cid, pl.ds(row, DCH)])
            pltpu.sync_copy(tab_d.at[pl.ds(row, DCH)], dbuf)
            pltpu.sync_copy(dbuf, out_hbm.at[1, cid, pl.ds(row, DCH)])

    return k(ones, src_ids, dst_ids, zeros)


def _const_spec(shape):
    return pl.BlockSpec(shape, lambda i: tuple(0 for _ in shape))


def _tc_prep(node_inputs, w_ih_n, bias):
    def body(x_ref, w_ref, b_ref, o_ref):
        o_ref[...] = (
            jnp.dot(x_ref[...], w_ref[...], preferred_element_type=_f32)
            + b_ref[...])

    return pl.pallas_call(
        body,
        grid=(NNB,),
        in_specs=[
            pl.BlockSpec((BNODE, F), lambda i: (i, 0)),
            _const_spec((F, 4 * F)),
            _const_spec((1, 4 * F)),
        ],
        out_specs=pl.BlockSpec((BNODE, 4 * F), lambda i: (i, 0)),
        out_shape=jax.ShapeDtypeStruct((N, 4 * F), _f32),
    )(node_inputs, w_ih_n, bias)


def _tc_node_stats(ns, wqk, bqk, cw):
    def body(ns_ref, w_ref, b_ref, cw_ref, s_ref):
        @pl.when(pl.program_id(0) == 0)
        def _():
            s_ref[...] = jnp.zeros_like(s_ref)

        r = (jnp.dot(ns_ref[...], w_ref[...], preferred_element_type=_f32)
             + b_ref[...])
        w = cw_ref[...]
        s_ref[0:1, :] += jnp.sum(w * r, axis=0, keepdims=True)
        s_ref[1:2, :] += jnp.sum(w * r * r, axis=0, keepdims=True)

    return pl.pallas_call(
        body,
        grid=(NNB,),
        in_specs=[
            pl.BlockSpec((BNODE, F), lambda i: (i, 0)),
            _const_spec((F, 2 * F)),
            _const_spec((1, 2 * F)),
            pl.BlockSpec((BNODE, 2 * F), lambda i: (i, 0)),
        ],
        out_specs=_const_spec((8, 2 * F)),
        out_shape=jax.ShapeDtypeStruct((8, 2 * F), _f32),
    )(ns, wqk, bqk, cw)


def _tc_edge1(gs, gd, wq, cq, wk, ck, w0s, w0d, b0, hsel):
    def body(gs_ref, gd_ref, wq_ref, cq_ref, wk_ref, ck_ref, w0s_ref,
             w0d_ref, b0_ref, hsel_ref, x_ref, sc_ref, st_ref):
        @pl.when(pl.program_id(0) == 0)
        def _():
            st_ref[...] = jnp.zeros_like(st_ref)
            st_ref[2:3, 0:8] = jnp.full((1, 8), -1e30, _f32)

        gsb = gs_ref[...]
        gdb = gd_ref[...]
        qh = jnp.dot(gsb, wq_ref[...], preferred_element_type=_f32) + cq_ref[...]
        kh = jnp.dot(gdb, wk_ref[...], preferred_element_type=_f32) + ck_ref[...]
        p = qh * kh
        s = jnp.dot(p, hsel_ref[...], preferred_element_type=_f32)
        s = jnp.where(s >= 0.0, s, 0.2 * s)
        sc_ref[...] = s
        x = (jnp.dot(gsb, w0s_ref[...], preferred_element_type=_f32)
             + jnp.dot(gdb, w0d_ref[...], preferred_element_type=_f32)
             + b0_ref[...])
        x_ref[...] = x
        st_ref[0:1, :] += jnp.sum(x, axis=0, keepdims=True)
        st_ref[1:2, :] += jnp.sum(x * x, axis=0, keepdims=True)
        st_ref[2:3, 0:8] = jnp.maximum(st_ref[2:3, 0:8],
                                       jnp.max(s, axis=0, keepdims=True))

    return pl.pallas_call(
        body,
        grid=(NEB,),
        in_specs=[
            pl.BlockSpec((BE, F), lambda i: (i, 0)),
            pl.BlockSpec((BE, F), lambda i: (i, 0)),
            _const_spec((F, F)),
            _const_spec((1, F)),
            _const_spec((F, F)),
            _const_spec((1, F)),
            _const_spec((F, M)),
            _const_spec((F, M)),
            _const_spec((1, M)),
            _const_spec((F, 8)),
        ],
        out_specs=[
            pl.BlockSpec((BE, M), lambda i: (i, 0)),
            pl.BlockSpec((BE, 8), lambda i: (i, 0)),
            _const_spec((8, M)),
        ],
        out_shape=[
            jax.ShapeDtypeStruct((E, M), _f32),
            jax.ShapeDtypeStruct((E, 8), _f32),
            jax.ShapeDtypeStruct((8, M), _f32),
        ],
    )(gs, gd, wq, cq, wk, ck, w0s, w0d, b0, hsel)


def _tc_edge_mid(xin, w, b, a0, b0n, score=None, gmax=None):
    with_ex = score is not None

    def body(*refs):
        if with_ex:
            (x_ref, sc_ref, w_ref, b_ref, a_ref, bn_ref, gm_ref,
             y_ref, ex_ref, st_ref) = refs
        else:
            x_ref, w_ref, b_ref, a_ref, bn_ref, y_ref, st_ref = refs

        @pl.when(pl.program_id(0) == 0)
        def _():
            st_ref[...] = jnp.zeros_like(st_ref)

        h = jnp.maximum(x_ref[...] * a_ref[...] + bn_ref[...], 0.0)
        y = jnp.dot(h, w_ref[...], preferred_element_type=_f32) + b_ref[...]
        y_ref[...] = y
        st_ref[0:1, :] += jnp.sum(y, axis=0, keepdims=True)
        st_ref[1:2, :] += jnp.sum(y * y, axis=0, keepdims=True)
        if with_ex:
            ex = jnp.exp(sc_ref[...] - gm_ref[...])
            lane = lax.broadcasted_iota(jnp.int32, (BE, 16), 1)
            ex_ref[...] = jnp.where(lane < HEADS,
                                    jnp.concatenate([ex, ex], axis=1), 0.0)

    in_specs = [pl.BlockSpec((BE, M), lambda i: (i, 0))]
    args = [xin]
    if with_ex:
        in_specs.append(pl.BlockSpec((BE, 8), lambda i: (i, 0)))
        args.append(score)
    in_specs += [_const_spec((M, M)), _const_spec((1, M)),
                 _const_spec((1, M)), _const_spec((1, M))]
    args += [w, b, a0, b0n]
    if with_ex:
        in_specs.append(_const_spec((1, 8)))
        args.append(gmax)
    out_specs = [pl.BlockSpec((BE, M), lambda i: (i, 0))]
    out_shape = [jax.ShapeDtypeStruct((E, M), _f32)]
    if with_ex:
        out_specs.append(pl.BlockSpec((BE, 16), lambda i: (i, 0)))
        out_shape.append(jax.ShapeDtypeStruct((E, 16), _f32))
    out_specs.append(_const_spec((8, M)))
    out_shape.append(jax.ShapeDtypeStruct((8, M), _f32))

    return pl.pallas_call(
        body,
        grid=(NEB,),
        in_specs=in_specs,
        out_specs=out_specs,
        out_shape=out_shape,
    )(*args)


def _tc_edge4(y2, ex16, rd, w3, b3, a2, b2n, expand):
    def body(y2_ref, ex_ref, rd_ref, w3_ref, b3_ref, a_ref, bn_ref,
             exp_ref, m_ref, st_ref):
        @pl.when(pl.program_id(0) == 0)
        def _():
            st_ref[...] = jnp.zeros_like(st_ref)

        h = jnp.maximum(y2_ref[...] * a_ref[...] + bn_ref[...], 0.0)
        y3 = jnp.dot(h, w3_ref[...], preferred_element_type=_f32) + b3_ref[...]
        attn = ex_ref[...] * rd_ref[...]
        attn128 = jnp.dot(attn, exp_ref[...], preferred_element_type=_f32)
        msg = y3 * attn128
        m_ref[...] = msg
        st_ref[0:1, :] += jnp.sum(msg, axis=0, keepdims=True)
        st_ref[1:2, :] += jnp.sum(msg * msg, axis=0, keepdims=True)

    return pl.pallas_call(
        body,
        grid=(NEB,),
        in_specs=[
            pl.BlockSpec((BE, M), lambda i: (i, 0)),
            pl.BlockSpec((BE, 16), lambda i: (i, 0)),
            pl.BlockSpec((BE, 16), lambda i: (i, 0)),
            _const_spec((M, TE)),
            _const_spec((1, TE)),
            _const_spec((1, M)),
            _const_spec((1, M)),
            _const_spec((16, TE)),
        ],
        out_specs=[
            pl.BlockSpec((BE, TE), lambda i: (i, 0)),
            _const_spec((8, TE)),
        ],
        out_shape=[
            jax.ShapeDtypeStruct((E, TE), _f32),
            jax.ShapeDtypeStruct((8, TE), _f32),
        ],
    )(y2, ex16, rd, w3, b3, a2, b2n, expand)


def _tc_node_update(p0, p1, cw, ns, gih, am, bm, w_ih_a, w_hh, wf, bf):
    def body(p0_ref, p1_ref, cw_ref, ns_ref, gih_ref, am_ref, bm_ref,
             wia_ref, whh_ref, wf_ref, bf_ref, ns_out, out_ref):
        cdb = cw_ref[:, F:]
        agg = (p0_ref[...] + p1_ref[...]) * am_ref[...] + cdb * bm_ref[...]
        nsb = ns_ref[...]
        gates = (jnp.dot(agg, wia_ref[...], preferred_element_type=_f32)
                 + gih_ref[...]
                 + jnp.dot(nsb, whh_ref[...], preferred_element_type=_f32))
        i_g = gates[:, 0:F]
        g_g = gates[:, 2 * F:3 * F]
        o_g = gates[:, 3 * F:4 * F]
        c = jax.nn.sigmoid(i_g) * jnp.tanh(g_g)
        h = jax.nn.sigmoid(o_g) * jnp.tanh(c)
        ns_new = h + nsb
        ns_out[...] = ns_new
        out_ref[...] = (jnp.dot(ns_new, wf_ref[...],
                                preferred_element_type=_f32) + bf_ref[...])

    return pl.pallas_call(
        body,
        grid=(NNB,),
        in_specs=[
            pl.BlockSpec((BNODE, TE), lambda i: (i, 0)),
            pl.BlockSpec((BNODE, TE), lambda i: (i, 0)),
            pl.BlockSpec((BNODE, 2 * F), lambda i: (i, 0)),
            pl.BlockSpec((BNODE, F), lambda i: (i, 0)),
            pl.BlockSpec((BNODE, 4 * F), lambda i: (i, 0)),
            _const_spec((1, TE)),
            _const_spec((1, TE)),
            _const_spec((TE, 4 * F)),
            _const_spec((F, 4 * F)),
            _const_spec((F, OUT)),
            _const_spec((1, OUT)),
        ],
        out_specs=[
            pl.BlockSpec((BNODE, F), lambda i: (i, 0)),
            pl.BlockSpec((BNODE, OUT), lambda i: (i, 0)),
        ],
        out_shape=[
            jax.ShapeDtypeStruct((N, F), _f32),
            jax.ShapeDtypeStruct((N, OUT), _f32),
        ],
    )(p0, p1, cw, ns, gih, am, bm, w_ih_a, w_hh, wf, bf)


def _realize(p, e):
    w = p["w_mu"] + jnp.exp(0.5 * p["w_lv"]) * e["w"]
    b = p["b_mu"] + jnp.exp(0.5 * p["b_lv"]) * e["b"]
    return w.T, b


def kernel(node_inputs, src_ids, dst_ids, params, eps):
    src_ids = src_ids.astype(jnp.int32)
    dst_ids = dst_ids.astype(jnp.int32)

    WqT, bq = _realize(params["q"], eps["q"])
    WkT, bk = _realize(params["k"], eps["k"])
    W0T, b0 = _realize(params["msg0"], eps["msg0"])
    W0s, W0d = W0T[:F], W0T[F:]
    W1T, b1 = _realize(params["msg1"], eps["msg1"])
    W2T, b2 = _realize(params["msg2"], eps["msg2"])
    W3T, b3 = _realize(params["msg3"], eps["msg3"])
    WfT, bf = _realize(params["final"], eps["final"])
    lp = params["lstm"]
    w_ih_a = lp["w_ih"][:, :TE].T
    w_ih_n = lp["w_ih"][:, TE:].T
    w_hhT = lp["w_hh"].T
    temp = jnp.clip(params["temp"], 0.5, 5.0)
    gq, bq2 = params["q_norm"]["g"], params["q_norm"]["b"]
    gk, bk2 = params["k_norm"]["g"], params["k_norm"]["b"]

    fidx = jnp.arange(F)[:, None]
    hsel = (fidx // HD == jnp.arange(8)[None, :]).astype(_f32)
    hidx = jnp.arange(16)[:, None]
    expand = (hidx == (jnp.arange(TE)[None, :] // HD)).astype(_f32)

    zeros_tab = jnp.zeros((DCH, TE), _f32)
    ones16 = jnp.ones((CH, 16), _f32)

    cnt = _sc_degree_counts(ones16, src_ids, dst_ids, zeros_tab[:, :16])
    cs = cnt[0, 0, :, 0] + cnt[0, 1, :, 0]
    cd = cnt[1, 0, :, 0] + cnt[1, 1, :, 0]
    cw = jnp.concatenate(
        [jnp.broadcast_to(cs[:, None], (N, F)),
         jnp.broadcast_to(cd[:, None], (N, F))], axis=1)

    gih = _tc_prep(node_inputs, w_ih_n, (lp["b_ih"] + lp["b_hh"])[None, :])

    wqk = jnp.concatenate([WqT, WkT], axis=1)
    bqk = jnp.concatenate([bq, bk])[None, :]

    ns = jnp.broadcast_to(params["init_emb"], (N, F))
    outs = []
    for _ in range(NITERS):
        st = _tc_node_stats(ns, wqk, bqk, cw)
        mqk = st[0] / E
        vqk = jnp.maximum(st[1] / E - mqk * mqk, 0.0)
        mq, mk = mqk[:F], mqk[F:]
        aq = gq / jnp.sqrt(vqk[:F] + BN_EPS)
        ak = gk / jnp.sqrt(vqk[F:] + BN_EPS)
        wq_eff = WqT * (aq / temp)[None, :]
        cq = (((bq - mq) * aq + bq2) / temp)[None, :]
        wk_eff = WkT * ak[None, :]
        ck = ((bk - mk) * ak + bk2)[None, :]

        gs, gd = _sc_gather_pair(ns, src_ids, dst_ids, F)

        x, score, st1 = _tc_edge1(gs, gd, wq_eff, cq, wk_eff, ck,
                                  W0s, W0d, b0[None, :], hsel)
        m0 = st1[0] / E
        v0 = jnp.maximum(st1[1] / E - m0 * m0, 0.0)
        a0 = params["mbn0"]["g"] / jnp.sqrt(v0 + BN_EPS)
        b0n = params["mbn0"]["b"] - m0 * a0
        gmax = st1[2:3, 0:8]

        y1, ex16, sty1 = _tc_edge_mid(x, W1T, b1[None, :], a0[None, :],
                                      b0n[None, :], score=score, gmax=gmax)
        m1 = sty1[0] / E
        v1 = jnp.maximum(sty1[1] / E - m1 * m1, 0.0)
        a1 = params["mbn1"]["g"] / jnp.sqrt(v1 + BN_EPS)
        b1n = params["mbn1"]["b"] - m1 * a1

        dpart = _sc_scatter_add(ex16, dst_ids, zeros_tab[:, :16], 16)
        denom = dpart[0] + dpart[1]
        rden = jnp.where(denom > 0.0, 1.0 / denom, 0.0)
        rd = _sc_gather(rden, dst_ids, 16)

        y2, sty2 = _tc_edge_mid(y1, W2T, b2[None, :], a1[None, :],
                                b1n[None, :])
        m2 = sty2[0] / E
        v2 = jnp.maximum(sty2[1] / E - m2 * m2, 0.0)
        a2 = params["mbn2"]["g"] / jnp.sqrt(v2 + BN_EPS)
        b2n = params["mbn2"]["b"] - m2 * a2

        msg, stm = _tc_edge4(y2, ex16, rd, W3T, b3[None, :], a2[None, :],
                             b2n[None, :], expand)
        mm = stm[0] / E
        vm = jnp.maximum(stm[1] / E - mm * mm, 0.0)
        am = params["msg_norm"]["g"] / jnp.sqrt(vm + BN_EPS)
        bmn = params["msg_norm"]["b"] - mm * am

        apart = _sc_scatter_add(msg, dst_ids, zeros_tab, TE)

        ns, out_i = _tc_node_update(apart[0], apart[1], cw, ns, gih,
                                    am[None, :], bmn[None, :],
                                    w_ih_a, w_hhT, WfT, bf[None, :])
        outs.append(out_i)

    return jnp.stack(outs, axis=0)


# speedup vs baseline: 1.6456x; 1.6456x over previous
import functools

import jax
import jax.numpy as jnp
from jax import lax
from jax.experimental import pallas as pl
from jax.experimental.pallas import tpu as pltpu
from jax.experimental.pallas import tpu_sc as plsc

N = 10000
E = 320000
F = 128
TE = 128
HEADS = 4
HD = 32
M = 256
OUT = 128
NITERS = 7
BN_EPS = 1e-5

NC = 2
NS = 16
NW = NC * NS
EPT = E // NW
CH = 400
NCHUNK = EPT // CH
NP = 10240
NP4 = NP * 4
TROWS = NP4 // 128
UR = E * 4 // 128
CHR = 80
NCHU = UR // CHR
EPTP = 10240
EPAD = EPTP * NW
PR = EPTP * 4 // 128
NCHP = PR // CHR

BE = 512
NEB = E // BE
BNODE = 400
NNB = N // BNODE

HTE = TE // 2
_f32 = jnp.float32


def _mesh():
    return plsc.VectorSubcoreMesh(core_axis_name="c", subcore_axis_name="s")


def _sc_gather_pair(table, ids_a, ids_b, width):
    @functools.partial(
        pl.kernel,
        out_type=(jax.ShapeDtypeStruct((E, width), _f32),
                  jax.ShapeDtypeStruct((E, width), _f32)),
        mesh=_mesh(),
        scratch_types=[
            pltpu.VMEM((CH,), jnp.int32),
            pltpu.VMEM((CH, width), _f32),
            pltpu.VMEM((CH,), jnp.int32),
            pltpu.VMEM((CH, width), _f32),
            pltpu.SemaphoreType.DMA,
            pltpu.SemaphoreType.DMA,
        ],
    )
    def k(tab_hbm, ia_hbm, ib_hbm, oa_hbm, ob_hbm, ia_v, ra_v, ib_v, rb_v,
          sa, sb):
        wid = lax.axis_index("s") * NC + lax.axis_index("c")
        base = wid * EPT

        @pl.loop(0, NCHUNK)
        def _(i):
            off = base + i * CH
            pltpu.sync_copy(ia_hbm.at[pl.ds(off, CH)], ia_v)
            pltpu.sync_copy(ib_hbm.at[pl.ds(off, CH)], ib_v)
            da = pltpu.async_copy(tab_hbm.at[ia_v], ra_v, sa)
            db = pltpu.async_copy(tab_hbm.at[ib_v], rb_v, sb)
            da.wait()
            db.wait()
            pltpu.sync_copy(ra_v, oa_hbm.at[pl.ds(off, CH)])
            pltpu.sync_copy(rb_v, ob_hbm.at[pl.ds(off, CH)])

    return k(table, ids_a, ids_b)


def _sc_segsum_feat(u_t, tgt):
    @functools.partial(
        pl.kernel,
        out_type=(jax.ShapeDtypeStruct((NW, TROWS, 128), _f32),
                  jax.ShapeDtypeStruct((NW, TROWS, 128), _f32)),
        mesh=_mesh(),
        compiler_params=pltpu.CompilerParams(needs_layout_passes=False),
        scratch_types=[
            pltpu.VMEM((CHR, 128), jnp.int32),
            pltpu.VMEM((CHR, 128), _f32),
            pltpu.VMEM((NP4,), _f32),
            pltpu.VMEM((NP4,), _f32),
        ],
    )
    def k(u_hbm, tgt_hbm, ou_hbm, ov_hbm, tbuf, ubuf, tab, tab2):
        wid = lax.axis_index("s") * NC + lax.axis_index("c")
        zeros = jnp.zeros((16,), _f32)

        @pl.loop(0, NP4 // 16)
        def _(v):
            tab[pl.ds(v * 16, 16)] = zeros
            tab2[pl.ds(v * 16, 16)] = zeros

        @pl.loop(0, NCHU)
        def _(i):
            pltpu.sync_copy(tgt_hbm.at[pl.ds(i * CHR, CHR)], tbuf)
            pltpu.sync_copy(u_hbm.at[wid, pl.ds(i * CHR, CHR)], ubuf)

            @pl.loop(0, CHR)
            def _(r):
                for j in range(8):
                    t = tbuf[r, pl.ds(j * 16, 16)]
                    v = ubuf[r, pl.ds(j * 16, 16)]
                    plsc.addupdate_scatter(tab, [t], v)
                    plsc.addupdate_scatter(tab2, [t], v * v)

        @pl.loop(0, TROWS)
        def _(r):
            pltpu.sync_copy(tab.at[pl.ds(r * 128, 128)], ou_hbm.at[wid, r])
            pltpu.sync_copy(tab2.at[pl.ds(r * 128, 128)], ov_hbm.at[wid, r])

    return k(u_t, tgt)


def _sc_segsum_edge(v_t, tgt_t):
    @functools.partial(
        pl.kernel,
        out_type=jax.ShapeDtypeStruct((NW, TROWS, 128), _f32),
        mesh=_mesh(),
        compiler_params=pltpu.CompilerParams(needs_layout_passes=False),
        scratch_types=[
            pltpu.VMEM((CHR, 128), jnp.int32),
            pltpu.VMEM((CHR, 128), _f32),
            pltpu.VMEM((NP4,), _f32),
        ],
    )
    def k(v_hbm, tgt_hbm, out_hbm, tbuf, ubuf, tab):
        wid = lax.axis_index("s") * NC + lax.axis_index("c")
        zeros = jnp.zeros((16,), _f32)

        @pl.loop(0, NP4 // 16)
        def _(v):
            tab[pl.ds(v * 16, 16)] = zeros

        @pl.loop(0, NCHP)
        def _(i):
            pltpu.sync_copy(tgt_hbm.at[wid, pl.ds(i * CHR, CHR)], tbuf)
            pltpu.sync_copy(v_hbm.at[wid, pl.ds(i * CHR, CHR)], ubuf)

            @pl.loop(0, CHR)
            def _(r):
                for j in range(8):
                    t = tbuf[r, pl.ds(j * 16, 16)]
                    v = ubuf[r, pl.ds(j * 16, 16)]
                    plsc.addupdate_scatter(tab, [t], v)

        @pl.loop(0, TROWS)
        def _(r):
            pltpu.sync_copy(tab.at[pl.ds(r * 128, 128)], out_hbm.at[wid, r])

    return k(v_t, tgt_t)


def _const_spec(shape):
    return pl.BlockSpec(shape, lambda i: tuple(0 for _ in shape))


def _tc_prep(node_inputs, w_ih_n, bias):
    def body(x_ref, w_ref, b_ref, o_ref):
        o_ref[...] = (
            jnp.dot(x_ref[...], w_ref[...], preferred_element_type=_f32)
            + b_ref[...])

    return pl.pallas_call(
        body,
        grid=(NNB,),
        in_specs=[
            pl.BlockSpec((BNODE, F), lambda i: (i, 0)),
            _const_spec((F, 4 * F)),
            _const_spec((1, 4 * F)),
        ],
        out_specs=pl.BlockSpec((BNODE, 4 * F), lambda i: (i, 0)),
        out_shape=jax.ShapeDtypeStruct((N, 4 * F), _f32),
    )(node_inputs, w_ih_n, bias)


def _tc_node_stats(ns, wqk, bqk, cw):
    def body(ns_ref, w_ref, b_ref, cw_ref, s_ref):
        @pl.when(pl.program_id(0) == 0)
        def _():
            s_ref[...] = jnp.zeros_like(s_ref)

        r = (jnp.dot(ns_ref[...], w_ref[...], preferred_element_type=_f32)
             + b_ref[...])
        w = cw_ref[...]
        s_ref[0:1, :] += jnp.sum(w * r, axis=0, keepdims=True)
        s_ref[1:2, :] += jnp.sum(w * r * r, axis=0, keepdims=True)

    return pl.pallas_call(
        body,
        grid=(NNB,),
        in_specs=[
            pl.BlockSpec((BNODE, F), lambda i: (i, 0)),
            _const_spec((F, 2 * F)),
            _const_spec((1, 2 * F)),
            pl.BlockSpec((BNODE, 2 * F), lambda i: (i, 0)),
        ],
        out_specs=_const_spec((8, 2 * F)),
        out_shape=jax.ShapeDtypeStruct((8, 2 * F), _f32),
    )(ns, wqk, bqk, cw)


def _tc_edge1(gs, gd, wq, cq, wk, ck, w0s, w0d, b0, hsel):
    def body(gs_ref, gd_ref, wq_ref, cq_ref, wk_ref, ck_ref, w0s_ref,
             w0d_ref, b0_ref, hsel_ref, x_ref, sc_ref, st_ref):
        @pl.when(pl.program_id(0) == 0)
        def _():
            st_ref[...] = jnp.zeros_like(st_ref)
            st_ref[2:3, 0:8] = jnp.full((1, 8), -1e30, _f32)

        gsb = gs_ref[...]
        gdb = gd_ref[...]
        qh = jnp.dot(gsb, wq_ref[...], preferred_element_type=_f32) + cq_ref[...]
        kh = jnp.dot(gdb, wk_ref[...], preferred_element_type=_f32) + ck_ref[...]
        p = qh * kh
        s = jnp.dot(p, hsel_ref[...], preferred_element_type=_f32,
                    precision=lax.Precision.HIGHEST)
        s = jnp.where(s >= 0.0, s, 0.2 * s)
        sc_ref[...] = s
        x = (jnp.dot(gsb, w0s_ref[...], preferred_element_type=_f32)
             + jnp.dot(gdb, w0d_ref[...], preferred_element_type=_f32)
             + b0_ref[...])
        x_ref[...] = x
        st_ref[0:1, :] += jnp.sum(x, axis=0, keepdims=True)
        st_ref[1:2, :] += jnp.sum(x * x, axis=0, keepdims=True)
        st_ref[2:3, 0:8] = jnp.maximum(st_ref[2:3, 0:8],
                                       jnp.max(s, axis=0, keepdims=True))

    return pl.pallas_call(
        body,
        grid=(NEB,),
        in_specs=[
            pl.BlockSpec((BE, F), lambda i: (i, 0)),
            pl.BlockSpec((BE, F), lambda i: (i, 0)),
            _const_spec((F, F)),
            _const_spec((1, F)),
            _const_spec((F, F)),
            _const_spec((1, F)),
            _const_spec((F, M)),
            _const_spec((F, M)),
            _const_spec((1, M)),
            _const_spec((F, 8)),
        ],
        out_specs=[
            pl.BlockSpec((BE, M), lambda i: (i, 0)),
            pl.BlockSpec((BE, 8), lambda i: (i, 0)),
            _const_spec((8, M)),
        ],
        out_shape=[
            jax.ShapeDtypeStruct((E, M), _f32),
            jax.ShapeDtypeStruct((E, 8), _f32),
            jax.ShapeDtypeStruct((8, M), _f32),
        ],
    )(gs, gd, wq, cq, wk, ck, w0s, w0d, b0, hsel)


def _tc_edge_mid(xin, w, b, a0, b0n, score=None, gmax=None):
    with_ex = score is not None

    def body(*refs):
        if with_ex:
            (x_ref, sc_ref, w_ref, b_ref, a_ref, bn_ref, gm_ref,
             y_ref, ex_ref, st_ref) = refs
        else:
            x_ref, w_ref, b_ref, a_ref, bn_ref, y_ref, st_ref = refs

        @pl.when(pl.program_id(0) == 0)
        def _():
            st_ref[...] = jnp.zeros_like(st_ref)

        h = jnp.maximum(x_ref[...] * a_ref[...] + bn_ref[...], 0.0)
        y = jnp.dot(h, w_ref[...], preferred_element_type=_f32) + b_ref[...]
        y_ref[...] = y
        st_ref[0:1, :] += jnp.sum(y, axis=0, keepdims=True)
        st_ref[1:2, :] += jnp.sum(y * y, axis=0, keepdims=True)
        if with_ex:
            ex = jnp.exp(sc_ref[...] - gm_ref[...])
            ex_ref[...] = ex[:, 0:HEADS]

    in_specs = [pl.BlockSpec((BE, M), lambda i: (i, 0))]
    args = [xin]
    if with_ex:
        in_specs.append(pl.BlockSpec((BE, 8), lambda i: (i, 0)))
        args.append(score)
    in_specs += [_const_spec((M, M)), _const_spec((1, M)),
                 _const_spec((1, M)), _const_spec((1, M))]
    args += [w, b, a0, b0n]
    if with_ex:
        in_specs.append(_const_spec((1, 8)))
        args.append(gmax)
    out_specs = [pl.BlockSpec((BE, M), lambda i: (i, 0))]
    out_shape = [jax.ShapeDtypeStruct((E, M), _f32)]
    if with_ex:
        out_specs.append(pl.BlockSpec((BE, HEADS), lambda i: (i, 0)))
        out_shape.append(jax.ShapeDtypeStruct((E, HEADS), _f32))
    out_specs.append(_const_spec((8, M)))
    out_shape.append(jax.ShapeDtypeStruct((8, M), _f32))

    return pl.pallas_call(
        body,
        grid=(NEB,),
        in_specs=in_specs,
        out_specs=out_specs,
        out_shape=out_shape,
    )(*args)


def _tc_edge4(y2, ex4, w3, b3, a2, b2n, expand):
    def body(y2_ref, ex_ref, w3_ref, b3_ref, a_ref, bn_ref, exp_ref, u_ref):
        h = jnp.maximum(y2_ref[...] * a_ref[...] + bn_ref[...], 0.0)
        y3 = jnp.dot(h, w3_ref[...], preferred_element_type=_f32) + b3_ref[...]
        ex128 = jnp.dot(ex_ref[...], exp_ref[...], preferred_element_type=_f32,
                        precision=lax.Precision.HIGHEST)
        u_ref[...] = y3 * ex128

    return pl.pallas_call(
        body,
        grid=(NEB,),
        in_specs=[
            pl.BlockSpec((BE, M), lambda i: (i, 0)),
            pl.BlockSpec((BE, HEADS), lambda i: (i, 0)),
            _const_spec((M, TE)),
            _const_spec((1, TE)),
            _const_spec((1, M)),
            _const_spec((1, M)),
            _const_spec((HEADS, TE)),
        ],
        out_specs=pl.BlockSpec((BE, TE), lambda i: (i, 0)),
        out_shape=jax.ShapeDtypeStruct((E, TE), _f32),
    )(y2, ex4, w3, b3, a2, b2n, expand)


def _tc_msg_stats(uu, vv, rd4, expand):
    def body(u_ref, v_ref, rd_ref, exp_ref, s_ref):
        @pl.when(pl.program_id(0) == 0)
        def _():
            s_ref[...] = jnp.zeros_like(s_ref)

        rh = jnp.dot(rd_ref[...], exp_ref[...], preferred_element_type=_f32,
                     precision=lax.Precision.HIGHEST)
        s_ref[0:1, :] += jnp.sum(rh * u_ref[...], axis=0, keepdims=True)
        s_ref[1:2, :] += jnp.sum(rh * rh * v_ref[...], axis=0, keepdims=True)

    return pl.pallas_call(
        body,
        grid=(NNB,),
        in_specs=[
            pl.BlockSpec((BNODE, TE), lambda i: (i, 0)),
            pl.BlockSpec((BNODE, TE), lambda i: (i, 0)),
            pl.BlockSpec((BNODE, HEADS), lambda i: (i, 0)),
            _const_spec((HEADS, TE)),
        ],
        out_specs=_const_spec((8, TE)),
        out_shape=jax.ShapeDtypeStruct((8, TE), _f32),
    )(uu, vv, rd4, expand)


def _tc_node_update(uu, rd4, expand, cw, ns, gih, am, bm, w_ih_a,
                    w_hh, wf, bf):
    def body(u_ref, rd_ref, exp_ref, cw_ref, ns_ref, gih_ref, am_ref,
             bm_ref, wia_ref, whh_ref, wf_ref, bf_ref, ns_out, out_ref):
        rh = jnp.dot(rd_ref[...], exp_ref[...], preferred_element_type=_f32,
                     precision=lax.Precision.HIGHEST)
        cdb = cw_ref[:, F:]
        agg = (rh * u_ref[...]) * am_ref[...] + cdb * bm_ref[...]
        nsb = ns_ref[...]
        gates = (jnp.dot(agg, wia_ref[...], preferred_element_type=_f32)
                 + gih_ref[...]
                 + jnp.dot(nsb, whh_ref[...], preferred_element_type=_f32))
        i_g = gates[:, 0:F]
        g_g = gates[:, 2 * F:3 * F]
        o_g = gates[:, 3 * F:4 * F]
        c = jax.nn.sigmoid(i_g) * jnp.tanh(g_g)
        h = jax.nn.sigmoid(o_g) * jnp.tanh(c)
        ns_new = h + nsb
        ns_out[...] = ns_new
        out_ref[...] = (jnp.dot(ns_new, wf_ref[...],
                                preferred_element_type=_f32) + bf_ref[...])

    return pl.pallas_call(
        body,
        grid=(NNB,),
        in_specs=[
            pl.BlockSpec((BNODE, TE), lambda i: (i, 0)),
            pl.BlockSpec((BNODE, HEADS), lambda i: (i, 0)),
            _const_spec((HEADS, TE)),
            pl.BlockSpec((BNODE, 2 * F), lambda i: (i, 0)),
            pl.BlockSpec((BNODE, F), lambda i: (i, 0)),
            pl.BlockSpec((BNODE, 4 * F), lambda i: (i, 0)),
            _const_spec((1, TE)),
            _const_spec((1, TE)),
            _const_spec((TE, 4 * F)),
            _const_spec((F, 4 * F)),
            _const_spec((F, OUT)),
            _const_spec((1, OUT)),
        ],
        out_specs=[
            pl.BlockSpec((BNODE, F), lambda i: (i, 0)),
            pl.BlockSpec((BNODE, OUT), lambda i: (i, 0)),
        ],
        out_shape=[
            jax.ShapeDtypeStruct((N, F), _f32),
            jax.ShapeDtypeStruct((N, OUT), _f32),
        ],
    )(uu, rd4, expand, cw, ns, gih, am, bm, w_ih_a, w_hh, wf, bf)


def _realize(p, e):
    w = p["w_mu"] + jnp.exp(0.5 * p["w_lv"]) * e["w"]
    b = p["b_mu"] + jnp.exp(0.5 * p["b_lv"]) * e["b"]
    return w.T, b


def kernel(node_inputs, src_ids, dst_ids, params, eps):
    src_ids = src_ids.astype(jnp.int32)
    dst_ids = dst_ids.astype(jnp.int32)

    WqT, bq = _realize(params["q"], eps["q"])
    WkT, bk = _realize(params["k"], eps["k"])
    W0T, b0 = _realize(params["msg0"], eps["msg0"])
    W0s, W0d = W0T[:F], W0T[F:]
    W1T, b1 = _realize(params["msg1"], eps["msg1"])
    W2T, b2 = _realize(params["msg2"], eps["msg2"])
    W3T, b3 = _realize(params["msg3"], eps["msg3"])
    WfT, bf = _realize(params["final"], eps["final"])
    lp = params["lstm"]
    w_ih_a = lp["w_ih"][:, :TE].T
    w_ih_n = lp["w_ih"][:, TE:].T
    w_hhT = lp["w_hh"].T
    temp = jnp.clip(params["temp"], 0.5, 5.0)
    gq, bq2 = params["q_norm"]["g"], params["q_norm"]["b"]
    gk, bk2 = params["k_norm"]["g"], params["k_norm"]["b"]

    fidx = jnp.arange(F)[:, None]
    hsel = (fidx // HD == jnp.arange(8)[None, :]).astype(_f32)
    hidx = jnp.arange(HEADS)[:, None]
    expand = (hidx == (jnp.arange(TE)[None, :] // HD)).astype(_f32)

    lane4 = jnp.arange(4, dtype=jnp.int32)[None, :]

    tgt_u = (dst_ids[:, None] * 4 + lane4).reshape(UR, 128)
    padN = EPAD - E
    dst_pad = jnp.concatenate(
        [dst_ids, jnp.full((padN,), NP - 1, jnp.int32)])
    src_pad = jnp.concatenate(
        [src_ids, jnp.full((padN,), NP - 1, jnp.int32)])
    tgt_d = (dst_pad[:, None] * 4 + lane4).reshape(NW, PR, 128)
    tgt_s = (src_pad[:, None] * 4 + lane4).reshape(NW, PR, 128)
    ones_pad = jnp.concatenate(
        [jnp.ones((E * 4,), _f32), jnp.zeros((padN * 4,), _f32)]
    ).reshape(NW, PR, 128)

    cs = _sc_segsum_edge(ones_pad, tgt_s)
    cs = jnp.sum(cs, axis=0).reshape(NP, 4)[:N, 0]
    cd = _sc_segsum_edge(ones_pad, tgt_d)
    cd = jnp.sum(cd, axis=0).reshape(NP, 4)[:N, 0]
    cw = jnp.concatenate(
        [jnp.broadcast_to(cs[:, None], (N, F)),
         jnp.broadcast_to(cd[:, None], (N, F))], axis=1)

    gih = _tc_prep(node_inputs, w_ih_n, (lp["b_ih"] + lp["b_hh"])[None, :])

    wqk = jnp.concatenate([WqT, WkT], axis=1)
    bqk = jnp.concatenate([bq, bk])[None, :]

    ns = jnp.broadcast_to(params["init_emb"], (N, F))
    outs = []
    for _ in range(NITERS):
        st = _tc_node_stats(ns, wqk, bqk, cw)
        mqk = st[0] / E
        vqk = jnp.maximum(st[1] / E - mqk * mqk, 0.0)
        mq, mk = mqk[:F], mqk[F:]
        aq = gq / jnp.sqrt(vqk[:F] + BN_EPS)
        ak = gk / jnp.sqrt(vqk[F:] + BN_EPS)
        wq_eff = WqT * (aq / temp)[None, :]
        cq = (((bq - mq) * aq + bq2) / temp)[None, :]
        wk_eff = WkT * ak[None, :]
        ck = ((bk - mk) * ak + bk2)[None, :]

        gs, gd = _sc_gather_pair(ns, src_ids, dst_ids, F)

        x, score, st1 = _tc_edge1(gs, gd, wq_eff, cq, wk_eff, ck,
                                  W0s, W0d, b0[None, :], hsel)
        m0 = st1[0] / E
        v0 = jnp.maximum(st1[1] / E - m0 * m0, 0.0)
        a0 = params["mbn0"]["g"] / jnp.sqrt(v0 + BN_EPS)
        b0n = params["mbn0"]["b"] - m0 * a0
        gmax = st1[2:3, 0:8]

        y1, ex4, sty1 = _tc_edge_mid(x, W1T, b1[None, :], a0[None, :],
                                     b0n[None, :], score=score, gmax=gmax)
        m1 = sty1[0] / E
        v1 = jnp.maximum(sty1[1] / E - m1 * m1, 0.0)
        a1 = params["mbn1"]["g"] / jnp.sqrt(v1 + BN_EPS)
        b1n = params["mbn1"]["b"] - m1 * a1

        ex_pad = jnp.concatenate(
            [ex4.reshape(E * 4), jnp.zeros((padN * 4,), _f32)]
        ).reshape(NW, PR, 128)
        dpart = _sc_segsum_edge(ex_pad, tgt_d)
        denom = jnp.sum(dpart, axis=0).reshape(NP, 4)[:N]
        rden = jnp.where(denom > 0.0, 1.0 / denom, 0.0)

        y2, sty2 = _tc_edge_mid(y1, W2T, b2[None, :], a1[None, :],
                                b1n[None, :])
        m2 = sty2[0] / E
        v2 = jnp.maximum(sty2[1] / E - m2 * m2, 0.0)
        a2 = params["mbn2"]["g"] / jnp.sqrt(v2 + BN_EPS)
        b2n = params["mbn2"]["b"] - m2 * a2

        u = _tc_edge4(y2, ex4, W3T, b3[None, :], a2[None, :],
                      b2n[None, :], expand)

        u_t = jnp.transpose(u.reshape(E, NW, 4), (1, 0, 2)).reshape(
            NW, UR, 128)
        up, vp = _sc_segsum_feat(u_t, tgt_u)
        uu = jnp.transpose(up.reshape(NW, NP, 4), (1, 0, 2)).reshape(
            NP, TE)[:N]
        vv = jnp.transpose(vp.reshape(NW, NP, 4), (1, 0, 2)).reshape(
            NP, TE)[:N]

        stm = _tc_msg_stats(uu, vv, rden, expand)
        mm = stm[0] / E
        vm = jnp.maximum(stm[1] / E - mm * mm, 0.0)
        am = params["msg_norm"]["g"] / jnp.sqrt(vm + BN_EPS)
        bmn = params["msg_norm"]["b"] - mm * am

        ns, out_i = _tc_node_update(uu, rden, expand, cw, ns, gih,
                                    am[None, :], bmn[None, :],
                                    w_ih_a, w_hhT, WfT, bf[None, :])
        outs.append(out_i)

    return jnp.stack(outs, axis=0)
